# baseline (speedup 1.0000x reference)
import functools

import jax
import jax.numpy as jnp
from jax.experimental import pallas as pl
from jax.experimental.pallas import tpu as pltpu

_PATCH = 16
_CIN = 3
_LANE = 128
_NB_CLASSES = 16
_VMEM_LIMIT_BYTES = 32 * 1024 * 1024


def _extract_kernel(x_ref, o_ref, *, batch, gw):
    cq = _CIN * _PATCH
    half = _PATCH // 2
    a = x_ref[...].reshape(batch, cq, gw * _PATCH)
    t = jnp.swapaxes(a, 1, 2)
    t = t.reshape(batch, gw, _PATCH, cq)
    lo = t[:, :, :half, :]
    hi = t[:, :, half:, :]
    pad = jnp.zeros((batch, gw, half, _LANE - 2 * cq), t.dtype)
    v = jnp.concatenate([lo, hi, pad], axis=-1)
    o_ref[...] = v.astype(jnp.bfloat16).reshape(batch, 1, gw, half, _LANE)


def _fused_kernel(p_ref, w_ref, b_ref, fcw_ref, sig_ref, feat_ref, log_ref,
                  *, np_per_img):
    h = jnp.dot(p_ref[...], w_ref[...], preferred_element_type=jnp.float32)
    h = jax.nn.gelu(h + b_ref[...])

    tm, _ = h.shape
    b_tile = tm // np_per_img
    col = jax.lax.broadcasted_iota(jnp.int32, (b_tile, tm), 1)
    row = jax.lax.broadcasted_iota(jnp.int32, (b_tile, tm), 0)
    pool = jnp.where(col // np_per_img == row,
                     jnp.float32(1.0 / np_per_img), jnp.float32(0.0))
    feats = jnp.dot(pool, h, preferred_element_type=jnp.float32)
    feat_ref[...] = feats

    xn = feats * jax.lax.rsqrt(
        jnp.maximum(jnp.sum(feats * feats, axis=-1, keepdims=True), 1e-24))
    logits = jnp.dot(xn, fcw_ref[...], preferred_element_type=jnp.float32)
    log_ref[...] = sig_ref[0, 0] * logits


def _forward(x, w_perm, branch_b_all, fc_wnT, fc_sigma):
    B, C, H, W = x.shape
    gh, gw = H // _PATCH, W // _PATCH
    np_per_img = gh * gw
    N = w_perm.shape[1]
    Cp = fc_wnT.shape[1]

    x5 = x.reshape(B, C, gh, _PATCH, W)
    extract = functools.partial(_extract_kernel, batch=B, gw=gw)
    half = _PATCH // 2
    patches5 = pl.pallas_call(
        extract,
        out_shape=jax.ShapeDtypeStruct((B, gh, gw, half, _LANE),
                                       jnp.bfloat16),
        grid_spec=pltpu.PrefetchScalarGridSpec(
            num_scalar_prefetch=0,
            grid=(gh,),
            in_specs=[pl.BlockSpec((B, C, 1, _PATCH, W),
                                   lambda i: (0, 0, i, 0, 0))],
            out_specs=pl.BlockSpec((B, 1, gw, half, _LANE),
                                   lambda i: (0, i, 0, 0, 0)),
        ),
        compiler_params=pltpu.CompilerParams(
            dimension_semantics=("parallel",),
            vmem_limit_bytes=_VMEM_LIMIT_BYTES),
    )(x5)
    Kp = half * _LANE
    patches = patches5.reshape(B * np_per_img, Kp)

    b_tile = 32
    tm = b_tile * np_per_img
    body = functools.partial(_fused_kernel, np_per_img=np_per_img)
    feats, logits_pad = pl.pallas_call(
        body,
        out_shape=(jax.ShapeDtypeStruct((B, N), jnp.float32),
                   jax.ShapeDtypeStruct((B, Cp), jnp.float32)),
        grid=(B // b_tile,),
        in_specs=[
            pl.BlockSpec((tm, Kp), lambda m: (m, 0)),
            pl.BlockSpec((Kp, N), lambda m: (0, 0)),
            pl.BlockSpec((1, N), lambda m: (0, 0)),
            pl.BlockSpec((N, Cp), lambda m: (0, 0)),
            pl.BlockSpec(memory_space=pltpu.MemorySpace.SMEM,
                         block_shape=(1, 1), index_map=lambda m: (0, 0)),
        ],
        out_specs=(pl.BlockSpec((b_tile, N), lambda m: (m, 0)),
                   pl.BlockSpec((b_tile, Cp), lambda m: (m, 0))),
        compiler_params=pltpu.CompilerParams(
            dimension_semantics=("parallel",),
            vmem_limit_bytes=_VMEM_LIMIT_BYTES),
    )(patches, w_perm, branch_b_all, fc_wnT, fc_sigma.reshape(1, 1))
    return feats, logits_pad


def kernel(x, branch_w_all, branch_b_all, proxy_wnT, proxy_sigma, fc_wnT,
           fc_sigma):
    del proxy_wnT, proxy_sigma
    B = x.shape[0]
    C = x.shape[1]
    N = branch_w_all.shape[1]
    half = _PATCH // 2
    cq = C * _PATCH

    w_r = (branch_w_all.reshape(C, _PATCH, _PATCH, N)
           .transpose(2, 0, 1, 3).reshape(_PATCH, cq, N))
    w_perm = jnp.concatenate(
        [w_r[:half], w_r[half:],
         jnp.zeros((half, _LANE - 2 * cq, N), w_r.dtype)], axis=1)
    w_perm = w_perm.reshape(half * _LANE, N)

    feats, logits_pad = _forward(x, w_perm, branch_b_all, fc_wnT, fc_sigma)
    return {'logits': logits_pad[:, :_NB_CLASSES], 'features': feats}


# speedup vs baseline: 1.0035x; 1.0035x over previous
import functools

import jax
import jax.numpy as jnp
from jax.experimental import pallas as pl
from jax.experimental.pallas import tpu as pltpu

_PATCH = 16
_CIN = 3
_LANE = 128
_NB_CLASSES = 16
_VMEM_LIMIT_BYTES = 32 * 1024 * 1024


def _extract_kernel(x_ref, o_ref, *, batch, gw):
    cq = _CIN * _PATCH
    half = _PATCH // 2
    a = x_ref[...].reshape(batch, cq, gw * _PATCH)
    t = jnp.swapaxes(a, 1, 2)
    t = t.reshape(batch, gw, _PATCH, cq)
    lo = t[:, :, :half, :]
    hi = t[:, :, half:, :]
    pad = jnp.zeros((batch, gw, half, _LANE - 2 * cq), t.dtype)
    v = jnp.concatenate([lo, hi, pad], axis=-1)
    o_ref[...] = v.astype(jnp.bfloat16).reshape(batch, 1, gw, half, _LANE)


def _fused_kernel(p_ref, w_ref, b_ref, fcw_ref, sig_ref, feat_ref, log_ref,
                  *, np_per_img):
    h = jnp.dot(p_ref[...], w_ref[...], preferred_element_type=jnp.float32)
    h = jax.nn.gelu(h + b_ref[...])

    tm, _ = h.shape
    b_tile = tm // np_per_img
    col = jax.lax.broadcasted_iota(jnp.int32, (b_tile, tm), 1)
    row = jax.lax.broadcasted_iota(jnp.int32, (b_tile, tm), 0)
    pool = jnp.where(col // np_per_img == row,
                     jnp.float32(1.0 / np_per_img), jnp.float32(0.0))
    feats = jnp.dot(pool, h, preferred_element_type=jnp.float32)
    feat_ref[...] = feats

    xn = feats * jax.lax.rsqrt(
        jnp.maximum(jnp.sum(feats * feats, axis=-1, keepdims=True), 1e-24))
    logits = jnp.dot(xn, fcw_ref[...], preferred_element_type=jnp.float32)
    log_ref[...] = sig_ref[0, 0] * logits


def _forward(x, w_perm, branch_b_all, fc_wnT, fc_sigma):
    B, C, H, W = x.shape
    gh, gw = H // _PATCH, W // _PATCH
    np_per_img = gh * gw
    N = w_perm.shape[1]
    Cp = fc_wnT.shape[1]

    x5 = x.reshape(B, C, gh, _PATCH, W)
    extract = functools.partial(_extract_kernel, batch=B, gw=gw)
    half = _PATCH // 2
    patches5 = pl.pallas_call(
        extract,
        out_shape=jax.ShapeDtypeStruct((B, gh, gw, half, _LANE),
                                       jnp.bfloat16),
        grid_spec=pltpu.PrefetchScalarGridSpec(
            num_scalar_prefetch=0,
            grid=(gh,),
            in_specs=[pl.BlockSpec((B, C, 1, _PATCH, W),
                                   lambda i: (0, 0, i, 0, 0))],
            out_specs=pl.BlockSpec((B, 1, gw, half, _LANE),
                                   lambda i: (0, i, 0, 0, 0)),
        ),
        compiler_params=pltpu.CompilerParams(
            dimension_semantics=("parallel",),
            vmem_limit_bytes=_VMEM_LIMIT_BYTES),
    )(x5)
    Kp = half * _LANE
    patches = patches5.reshape(B * np_per_img, Kp)

    b_tile = min(32, B)
    tm = b_tile * np_per_img
    body = functools.partial(_fused_kernel, np_per_img=np_per_img)
    feats, logits_pad = pl.pallas_call(
        body,
        out_shape=(jax.ShapeDtypeStruct((B, N), jnp.float32),
                   jax.ShapeDtypeStruct((B, Cp), jnp.float32)),
        grid=(B // b_tile,),
        in_specs=[
            pl.BlockSpec((tm, Kp), lambda m: (m, 0)),
            pl.BlockSpec((Kp, N), lambda m: (0, 0)),
            pl.BlockSpec((1, N), lambda m: (0, 0)),
            pl.BlockSpec((N, Cp), lambda m: (0, 0)),
            pl.BlockSpec(memory_space=pltpu.MemorySpace.SMEM,
                         block_shape=(1, 1), index_map=lambda m: (0, 0)),
        ],
        out_specs=(pl.BlockSpec((b_tile, N), lambda m: (m, 0)),
                   pl.BlockSpec((b_tile, Cp), lambda m: (m, 0))),
        compiler_params=pltpu.CompilerParams(
            dimension_semantics=("parallel",),
            vmem_limit_bytes=_VMEM_LIMIT_BYTES),
    )(patches, w_perm, branch_b_all, fc_wnT, fc_sigma.reshape(1, 1))
    return feats, logits_pad


def kernel(x, branch_w_all, branch_b_all, proxy_wnT, proxy_sigma, fc_wnT,
           fc_sigma):
    del proxy_wnT, proxy_sigma
    B = x.shape[0]
    C = x.shape[1]
    N = branch_w_all.shape[1]
    half = _PATCH // 2
    cq = C * _PATCH

    w_r = (branch_w_all.reshape(C, _PATCH, _PATCH, N)
           .transpose(2, 0, 1, 3).reshape(_PATCH, cq, N))
    w_perm = jnp.concatenate(
        [w_r[:half], w_r[half:],
         jnp.zeros((half, _LANE - 2 * cq, N), w_r.dtype)], axis=1)
    w_perm = w_perm.reshape(half * _LANE, N)

    feats, logits_pad = _forward(x, w_perm, branch_b_all, fc_wnT, fc_sigma)
    return {'logits': logits_pad[:, :_NB_CLASSES], 'features': feats}


# speedup vs baseline: 1.0445x; 1.0409x over previous
import functools

import jax
import jax.numpy as jnp
from jax.experimental import pallas as pl
from jax.experimental.pallas import tpu as pltpu

_PATCH = 16
_CIN = 3
_LANE = 128
_NB_CLASSES = 16
_VMEM_LIMIT_BYTES = 32 * 1024 * 1024


def _extract_kernel(x_ref, o_ref, *, batch, gw, ipb):
    cq = _CIN * _PATCH
    half = _PATCH // 2
    for u in range(ipb):
        a = x_ref[:, :, u].reshape(batch, cq, gw * _PATCH)
        t = jnp.swapaxes(a, 1, 2)
        t = t.reshape(batch, gw, _PATCH, cq)
        lo = t[:, :, :half, :]
        hi = t[:, :, half:, :]
        pad = jnp.zeros((batch, gw, half, _LANE - 2 * cq), t.dtype)
        v = jnp.concatenate([lo, hi, pad], axis=-1)
        o_ref[:, u] = v.astype(jnp.bfloat16)


def _fused_kernel(p_ref, w_ref, b_ref, fcw_ref, sig_ref, feat_ref, log_ref,
                  *, np_per_img):
    h = jnp.dot(p_ref[...], w_ref[...], preferred_element_type=jnp.float32)
    h = jax.nn.gelu(h + b_ref[...])

    tm, _ = h.shape
    b_tile = tm // np_per_img
    col = jax.lax.broadcasted_iota(jnp.int32, (b_tile, tm), 1)
    row = jax.lax.broadcasted_iota(jnp.int32, (b_tile, tm), 0)
    pool = jnp.where(col // np_per_img == row,
                     jnp.float32(1.0 / np_per_img), jnp.float32(0.0))
    feats = jnp.dot(pool, h, preferred_element_type=jnp.float32)
    feat_ref[...] = feats

    xn = feats * jax.lax.rsqrt(
        jnp.maximum(jnp.sum(feats * feats, axis=-1, keepdims=True), 1e-24))
    logits = jnp.dot(xn, fcw_ref[...], preferred_element_type=jnp.float32)
    log_ref[...] = sig_ref[0, 0] * logits


def _forward(x, w_perm, branch_b_all, fc_wnT, fc_sigma):
    B, C, H, W = x.shape
    gh, gw = H // _PATCH, W // _PATCH
    np_per_img = gh * gw
    N = w_perm.shape[1]
    Cp = fc_wnT.shape[1]

    x5 = x.reshape(B, C, gh, _PATCH, W)
    ipb = 2 if gh % 2 == 0 else 1
    extract = functools.partial(_extract_kernel, batch=B, gw=gw, ipb=ipb)
    half = _PATCH // 2
    patches5 = pl.pallas_call(
        extract,
        out_shape=jax.ShapeDtypeStruct((B, gh, gw, half, _LANE),
                                       jnp.bfloat16),
        grid_spec=pltpu.PrefetchScalarGridSpec(
            num_scalar_prefetch=0,
            grid=(gh // ipb,),
            in_specs=[pl.BlockSpec((B, C, ipb, _PATCH, W),
                                   lambda i: (0, 0, i, 0, 0))],
            out_specs=pl.BlockSpec((B, ipb, gw, half, _LANE),
                                   lambda i: (0, i, 0, 0, 0)),
        ),
        compiler_params=pltpu.CompilerParams(
            dimension_semantics=("parallel",),
            vmem_limit_bytes=_VMEM_LIMIT_BYTES),
    )(x5)
    Kp = half * _LANE
    patches = patches5.reshape(B * np_per_img, Kp)

    b_tile = min(32, B)
    tm = b_tile * np_per_img
    body = functools.partial(_fused_kernel, np_per_img=np_per_img)
    feats, logits_pad = pl.pallas_call(
        body,
        out_shape=(jax.ShapeDtypeStruct((B, N), jnp.float32),
                   jax.ShapeDtypeStruct((B, Cp), jnp.float32)),
        grid=(B // b_tile,),
        in_specs=[
            pl.BlockSpec((tm, Kp), lambda m: (m, 0)),
            pl.BlockSpec((Kp, N), lambda m: (0, 0)),
            pl.BlockSpec((1, N), lambda m: (0, 0)),
            pl.BlockSpec((N, Cp), lambda m: (0, 0)),
            pl.BlockSpec(memory_space=pltpu.MemorySpace.SMEM,
                         block_shape=(1, 1), index_map=lambda m: (0, 0)),
        ],
        out_specs=(pl.BlockSpec((b_tile, N), lambda m: (m, 0)),
                   pl.BlockSpec((b_tile, Cp), lambda m: (m, 0))),
        compiler_params=pltpu.CompilerParams(
            dimension_semantics=("parallel",),
            vmem_limit_bytes=_VMEM_LIMIT_BYTES),
    )(patches, w_perm, branch_b_all, fc_wnT, fc_sigma.reshape(1, 1))
    return feats, logits_pad


def kernel(x, branch_w_all, branch_b_all, proxy_wnT, proxy_sigma, fc_wnT,
           fc_sigma):
    del proxy_wnT, proxy_sigma
    B = x.shape[0]
    C = x.shape[1]
    N = branch_w_all.shape[1]
    half = _PATCH // 2
    cq = C * _PATCH

    w_r = (branch_w_all.reshape(C, _PATCH, _PATCH, N)
           .transpose(2, 0, 1, 3).reshape(_PATCH, cq, N))
    w_perm = jnp.concatenate(
        [w_r[:half], w_r[half:],
         jnp.zeros((half, _LANE - 2 * cq, N), w_r.dtype)], axis=1)
    w_perm = w_perm.reshape(half * _LANE, N)

    feats, logits_pad = _forward(x, w_perm, branch_b_all, fc_wnT, fc_sigma)
    return {'logits': logits_pad[:, :_NB_CLASSES], 'features': feats}


# speedup vs baseline: 1.0682x; 1.0226x over previous
import functools

import jax
import jax.numpy as jnp
from jax.experimental import pallas as pl
from jax.experimental.pallas import tpu as pltpu

_PATCH = 16
_CIN = 3
_LANE = 128
_NB_CLASSES = 16
_VMEM_LIMIT_BYTES = 32 * 1024 * 1024


def _extract_kernel(x_ref, o_ref, *, batch, gw, ipb):
    cq = _CIN * _PATCH
    half = _PATCH // 2
    for u in range(ipb):
        a = x_ref[:, :, u].reshape(batch, cq, gw * _PATCH)
        t = jnp.swapaxes(a, 1, 2)
        t = t.reshape(batch, gw, _PATCH, cq)
        lo = t[:, :, :half, :]
        hi = t[:, :, half:, :]
        pad = jnp.zeros((batch, gw, half, _LANE - 2 * cq), t.dtype)
        v = jnp.concatenate([lo, hi, pad], axis=-1)
        o_ref[:, u] = v.astype(jnp.bfloat16)


def _fused_kernel(p_ref, w_ref, b_ref, fcw_ref, sig_ref, feat_ref, log_ref,
                  *, np_per_img):
    h = jnp.dot(p_ref[...], w_ref[...], preferred_element_type=jnp.float32)
    h = jax.nn.gelu(h + b_ref[...])

    tm, _ = h.shape
    b_tile = tm // np_per_img
    col = jax.lax.broadcasted_iota(jnp.int32, (b_tile, tm), 1)
    row = jax.lax.broadcasted_iota(jnp.int32, (b_tile, tm), 0)
    pool = jnp.where(col // np_per_img == row,
                     jnp.float32(1.0 / np_per_img), jnp.float32(0.0))
    feats = jnp.dot(pool, h, preferred_element_type=jnp.float32)
    feat_ref[...] = feats

    xn = feats * jax.lax.rsqrt(
        jnp.maximum(jnp.sum(feats * feats, axis=-1, keepdims=True), 1e-24))
    logits = jnp.dot(xn, fcw_ref[...], preferred_element_type=jnp.float32)
    log_ref[...] = sig_ref[0, 0] * logits


def _forward(x, w_perm, branch_b_all, fc_wnT, fc_sigma):
    B, C, H, W = x.shape
    gh, gw = H // _PATCH, W // _PATCH
    np_per_img = gh * gw
    N = w_perm.shape[1]
    Cp = fc_wnT.shape[1]

    x5 = x.reshape(B, C, gh, _PATCH, W)
    ipb = 7 if gh % 7 == 0 else (2 if gh % 2 == 0 else 1)
    extract = functools.partial(_extract_kernel, batch=B, gw=gw, ipb=ipb)
    half = _PATCH // 2
    patches5 = pl.pallas_call(
        extract,
        out_shape=jax.ShapeDtypeStruct((B, gh, gw, half, _LANE),
                                       jnp.bfloat16),
        grid_spec=pltpu.PrefetchScalarGridSpec(
            num_scalar_prefetch=0,
            grid=(gh // ipb,),
            in_specs=[pl.BlockSpec((B, C, ipb, _PATCH, W),
                                   lambda i: (0, 0, i, 0, 0))],
            out_specs=pl.BlockSpec((B, ipb, gw, half, _LANE),
                                   lambda i: (0, i, 0, 0, 0)),
        ),
        compiler_params=pltpu.CompilerParams(
            dimension_semantics=("parallel",),
            vmem_limit_bytes=48 * 1024 * 1024),
    )(x5)
    Kp = half * _LANE
    patches = patches5.reshape(B * np_per_img, Kp)

    b_tile = min(32, B)
    tm = b_tile * np_per_img
    body = functools.partial(_fused_kernel, np_per_img=np_per_img)
    feats, logits_pad = pl.pallas_call(
        body,
        out_shape=(jax.ShapeDtypeStruct((B, N), jnp.float32),
                   jax.ShapeDtypeStruct((B, Cp), jnp.float32)),
        grid=(B // b_tile,),
        in_specs=[
            pl.BlockSpec((tm, Kp), lambda m: (m, 0)),
            pl.BlockSpec((Kp, N), lambda m: (0, 0)),
            pl.BlockSpec((1, N), lambda m: (0, 0)),
            pl.BlockSpec((N, Cp), lambda m: (0, 0)),
            pl.BlockSpec(memory_space=pltpu.MemorySpace.SMEM,
                         block_shape=(1, 1), index_map=lambda m: (0, 0)),
        ],
        out_specs=(pl.BlockSpec((b_tile, N), lambda m: (m, 0)),
                   pl.BlockSpec((b_tile, Cp), lambda m: (m, 0))),
        compiler_params=pltpu.CompilerParams(
            dimension_semantics=("parallel",),
            vmem_limit_bytes=_VMEM_LIMIT_BYTES),
    )(patches, w_perm, branch_b_all, fc_wnT, fc_sigma.reshape(1, 1))
    return feats, logits_pad


def kernel(x, branch_w_all, branch_b_all, proxy_wnT, proxy_sigma, fc_wnT,
           fc_sigma):
    del proxy_wnT, proxy_sigma
    B = x.shape[0]
    C = x.shape[1]
    N = branch_w_all.shape[1]
    half = _PATCH // 2
    cq = C * _PATCH

    w_r = (branch_w_all.reshape(C, _PATCH, _PATCH, N)
           .transpose(2, 0, 1, 3).reshape(_PATCH, cq, N))
    w_perm = jnp.concatenate(
        [w_r[:half], w_r[half:],
         jnp.zeros((half, _LANE - 2 * cq, N), w_r.dtype)], axis=1)
    w_perm = w_perm.reshape(half * _LANE, N)

    feats, logits_pad = _forward(x, w_perm, branch_b_all, fc_wnT, fc_sigma)
    return {'logits': logits_pad[:, :_NB_CLASSES], 'features': feats}
